# Initial kernel scaffold; baseline (speedup 1.0000x reference)
import jax
import jax.numpy as jnp
from jax.experimental import pallas as pl


def kernel(indices, probs):
    raise NotImplementedError("write your pallas kernel here")


# speedup vs baseline: 1.3066x; 1.3066x over previous
import functools

import jax
import jax.numpy as jnp
from jax import lax
from jax.experimental import pallas as pl
from jax.experimental.pallas import tpu as pltpu
from jax.experimental.pallas import tpu_sc as plsc

NUM_CLASSES = 1_000_000
NC = 2
NS = 16
L = 16
NW = NC * NS

CHUNK = 62_720
SUB = CHUNK // 8
VPAD = NS * CHUNK
B_TOT = 4096 * 200
ROWS = B_TOT // 128
ROWS_W = ROWS // NW

_LN2 = 0.6931471805599453


def _log_f32(v):
    bits = lax.bitcast_convert_type(v, jnp.int32)
    e = lax.shift_right_arithmetic(bits, 23) - 127
    m_bits = lax.bitwise_or(lax.bitwise_and(bits, 0x007FFFFF), 0x3F800000)
    m = lax.bitcast_convert_type(m_bits, jnp.float32)
    r = (m - 1.0) / (m + 1.0)
    r2 = r * r
    lnm = r * (2.0 + r2 * (0.6666666666666666
                           + r2 * (0.4 + r2 * 0.2857142857142857)))
    return lnm + e.astype(jnp.float32) * _LN2


def _body(idx_hbm, probs_hbm, out_hbm,
          table_sp, part_sp, chunk_v, idx_v, vals_v, pvt_v, pall_v, sem):
    cid = lax.axis_index("c")
    sid = lax.axis_index("s")
    wid = sid * NC + cid

    base = sid * CHUNK
    acc = jnp.zeros((L,), jnp.float32)
    for k in range(CHUNK // SUB):
        pltpu.sync_copy(probs_hbm.at[pl.ds(base + k * SUB, SUB)], chunk_v)
        pltpu.sync_copy(chunk_v, table_sp.at[pl.ds(base + k * SUB, SUB)])

        def sum_step(i, a):
            return a + chunk_v[pl.ds(i * L, L)]
        acc = lax.fori_loop(0, SUB // L, sum_step, acc)
    pvt_v[...] = acc
    pltpu.sync_copy(pvt_v, part_sp.at[sid])

    plsc.subcore_barrier()

    pltpu.sync_copy(part_sp, pall_v)
    tot = pall_v[0]
    for t in range(1, NS):
        tot = tot + pall_v[t]
    s = tot[0]
    for i in range(1, L):
        s = s + tot[i]
    ln_s = _log_f32(jnp.full((L,), s, jnp.float32))

    pltpu.sync_copy(idx_hbm.at[pl.ds(wid * ROWS_W, ROWS_W)], idx_v)

    def fire(r, carry):
        pltpu.async_copy(table_sp.at[idx_v.at[r]], vals_v.at[r], sem)
        return carry
    lax.fori_loop(0, ROWS_W, fire, 0)
    pltpu.make_async_copy(
        out_hbm.at[pl.ds(wid * ROWS_W, ROWS_W)], vals_v, sem).wait()

    def log_row(r, carry):
        for c in range(128 // L):
            v = vals_v[r, pl.ds(c * L, L)]
            vals_v[r, pl.ds(c * L, L)] = _log_f32(v) - ln_s
        return carry
    lax.fori_loop(0, ROWS_W, log_row, 0)

    pltpu.sync_copy(vals_v, out_hbm.at[pl.ds(wid * ROWS_W, ROWS_W)])


@jax.jit
def kernel(indices, probs):
    idx2d = indices.reshape(ROWS, 128)
    probs_pad = jnp.concatenate(
        [probs, jnp.zeros((VPAD - NUM_CLASSES,), jnp.float32)])
    mesh = plsc.VectorSubcoreMesh(core_axis_name="c", subcore_axis_name="s")
    fn = pl.kernel(
        _body,
        out_type=jax.ShapeDtypeStruct((ROWS, 128), jnp.float32),
        mesh=mesh,
        scratch_types=[
            pltpu.VMEM_SHARED((VPAD,), jnp.float32),
            pltpu.VMEM_SHARED((NS, L), jnp.float32),
            pltpu.VMEM((SUB,), jnp.float32),
            pltpu.VMEM((ROWS_W, 128), jnp.int32),
            pltpu.VMEM((ROWS_W, 128), jnp.float32),
            pltpu.VMEM((L,), jnp.float32),
            pltpu.VMEM((NS, L), jnp.float32),
            pltpu.SemaphoreType.DMA,
        ],
    )
    out = fn(idx2d, probs_pad)
    return out.reshape(4096, 200)


# speedup vs baseline: 1.5683x; 1.2003x over previous
import jax
import jax.numpy as jnp
from jax import lax
from jax.experimental import pallas as pl
from jax.experimental.pallas import tpu as pltpu
from jax.experimental.pallas import tpu_sc as plsc

NUM_CLASSES = 1_000_000
NC = 2
NS = 16
L = 16
NW = NC * NS

CHUNK = 62_720
NK = 16
SUB = CHUNK // NK
SUM_U = 7
VPAD = NS * CHUNK
B_TOT = 4096 * 200
ROWS = B_TOT // 128
ROWS_W = ROWS // NW
G = 40
NG = ROWS_W // G

_LN2 = 0.6931471805599453
_P5 = (0.0439291, -0.40947993, 1.61019286, -3.52024492, 5.06977785, -2.7941606)


def _log_from_bits(v, ln_s):
    bits = lax.bitcast_convert_type(v, jnp.int32)
    e = lax.shift_right_arithmetic(bits, 23) - 127
    m_bits = lax.bitwise_or(lax.bitwise_and(bits, 0x007FFFFF), 0x3F800000)
    m = lax.bitcast_convert_type(m_bits, jnp.float32)
    p = jnp.float32(_P5[0])
    for c in _P5[1:]:
        p = p * m + jnp.float32(c)
    return (e.astype(jnp.float32) + p) * jnp.float32(_LN2) - ln_s


def _body(idx_hbm, probs_hbm, out_hbm,
          table_sp, part_sp, buf0, buf1, idx_v, vals_v, pvt_v, pall_v,
          sem_h0, sem_h1, sem_s0, sem_s1, sem_idx, sem_g0, sem_g1, sem_out):
    cid = lax.axis_index("c")
    sid = lax.axis_index("s")
    wid = sid * NC + cid
    base = sid * CHUNK
    obase = wid * ROWS_W

    h_idx = pltpu.async_copy(idx_hbm.at[pl.ds(obase, ROWS_W)], idx_v, sem_idx)

    bufs = (buf0, buf1)
    accs = [jnp.zeros((L,), jnp.float32) for _ in range(4)]
    for k in range(NK):
        b = k % 2
        buf = bufs[b]
        pltpu.sync_copy(probs_hbm.at[pl.ds(base + k * SUB, SUB)], buf)
        pltpu.sync_copy(buf, table_sp.at[pl.ds(base + k * SUB, SUB)])

        def sum_step(i, a):
            o = i * (SUM_U * L)
            for u in range(SUM_U):
                a = tuple(
                    a[j] + buf[pl.ds(o + u * L, L)] if j == (u % 4) else a[j]
                    for j in range(4))
            return a
        accs = list(lax.fori_loop(0, SUB // (SUM_U * L), sum_step, tuple(accs)))
    acc = (accs[0] + accs[1]) + (accs[2] + accs[3])

    pvt_v[...] = acc
    pltpu.sync_copy(pvt_v, part_sp.at[sid])

    plsc.subcore_barrier()

    h_idx.wait()
    sems_g = (sem_g0, sem_g1)

    def fire_group(g):
        s = sems_g[g % 2]

        def fire(r, c):
            row = g * G + r
            pltpu.async_copy(table_sp.at[idx_v.at[row]], vals_v.at[row], s)
            return c
        lax.fori_loop(0, G, fire, 0)

    fire_group(0)
    fire_group(1)

    pltpu.sync_copy(part_sp, pall_v)
    tot = pall_v[0]
    for t in range(1, NS):
        tot = tot + pall_v[t]
    s_scalar = tot[0]
    for i in range(1, L):
        s_scalar = s_scalar + tot[i]
    ln_s = _log_from_bits(jnp.full((L,), s_scalar, jnp.float32),
                          jnp.zeros((L,), jnp.float32))

    for g in range(NG):
        pltpu.make_async_copy(
            out_hbm.at[pl.ds(obase + g * G, G)],
            vals_v.at[pl.ds(g * G, G)], sems_g[g % 2]).wait()
        if g + 2 < NG:
            fire_group(g + 2)

        def log_row(r, carry):
            row = g * G + r
            for c in range(128 // L):
                v = vals_v[row, pl.ds(c * L, L)]
                vals_v[row, pl.ds(c * L, L)] = _log_from_bits(v, ln_s)
            return carry
        lax.fori_loop(0, G, log_row, 0)

        pltpu.async_copy(vals_v.at[pl.ds(g * G, G)],
                         out_hbm.at[pl.ds(obase + g * G, G)], sem_out)

    pltpu.make_async_copy(out_hbm.at[pl.ds(obase, ROWS_W)],
                          vals_v, sem_out).wait()


@jax.jit
def kernel(indices, probs):
    idx2d = indices.reshape(ROWS, 128)
    probs_pad = jnp.concatenate(
        [probs, jnp.zeros((VPAD - NUM_CLASSES,), jnp.float32)])
    mesh = plsc.VectorSubcoreMesh(core_axis_name="c", subcore_axis_name="s")
    fn = pl.kernel(
        _body,
        out_type=jax.ShapeDtypeStruct((ROWS, 128), jnp.float32),
        mesh=mesh,
        scratch_types=[
            pltpu.VMEM_SHARED((VPAD,), jnp.float32),
            pltpu.VMEM_SHARED((NS, L), jnp.float32),
            pltpu.VMEM((SUB,), jnp.float32),
            pltpu.VMEM((SUB,), jnp.float32),
            pltpu.VMEM((ROWS_W, 128), jnp.int32),
            pltpu.VMEM((ROWS_W, 128), jnp.float32),
            pltpu.VMEM((L,), jnp.float32),
            pltpu.VMEM((NS, L), jnp.float32),
        ] + [pltpu.SemaphoreType.DMA] * 8,
    )
    out = fn(idx2d, probs_pad)
    return out.reshape(4096, 200)


# speedup vs baseline: 1.7129x; 1.0922x over previous
import jax
import jax.numpy as jnp
from jax import lax
from jax.experimental import pallas as pl
from jax.experimental.pallas import tpu as pltpu
from jax.experimental.pallas import tpu_sc as plsc

NUM_CLASSES = 1_000_000
NC = 2
NS = 16
L = 16
NW = NC * NS

CHUNK = 62_976
NWAVE = 16
WSET = 2
SUB = CHUNK // (NWAVE * WSET)
SET = WSET * SUB
VPAD = NS * CHUNK
B_TOT = 4096 * 200
ROWS = B_TOT // 128
ROWS_W = ROWS // NW
G = 40
NG = ROWS_W // G

_LN2 = 0.6931471805599453
_P5 = (0.0439291, -0.40947993, 1.61019286, -3.52024492, 5.06977785, -2.7941606)


def _log_from_bits(v, ln_s):
    bits = lax.bitcast_convert_type(v, jnp.int32)
    e = lax.shift_right_arithmetic(bits, 23) - 127
    m_bits = lax.bitwise_or(lax.bitwise_and(bits, 0x007FFFFF), 0x3F800000)
    m = lax.bitcast_convert_type(m_bits, jnp.float32)
    p = jnp.float32(_P5[0])
    for c in _P5[1:]:
        p = p * m + jnp.float32(c)
    return (e.astype(jnp.float32) + p) * jnp.float32(_LN2) - ln_s


def _body(idx_hbm, probs_hbm, out_hbm,
          table_sp, part_sp, buf_a, buf_b, idx_v, vals_v, pvt_v, pall_v,
          sem_a, sem_b, sem_stage, sem_idx, sem_g0, sem_g1, sem_out):
    cid = lax.axis_index("c")
    sid = lax.axis_index("s")
    wid = sid * NC + cid
    base = sid * CHUNK
    obase = wid * ROWS_W

    h_idx = pltpu.async_copy(idx_hbm.at[pl.ds(obase, ROWS_W)], idx_v, sem_idx)

    h_stage = pltpu.async_copy(
        probs_hbm.at[pl.ds(base, CHUNK)],
        table_sp.at[pl.ds(base, CHUNK)], sem_stage)

    sets = (buf_a, buf_b)
    sems = (sem_a, sem_b)

    def fire_wave(w):
        bset = sets[w % 2]
        for j in range(WSET):
            pltpu.async_copy(
                probs_hbm.at[pl.ds(base + (w * WSET + j) * SUB, SUB)],
                bset.at[pl.ds(j * SUB, SUB)], sems[w % 2])

    fire_wave(0)
    accs = [jnp.zeros((L,), jnp.float32) for _ in range(4)]
    for w in range(NWAVE):
        if w + 1 < NWAVE:
            fire_wave(w + 1)
        bset = sets[w % 2]
        pltpu.make_async_copy(
            probs_hbm.at[pl.ds(base, SET)], bset, sems[w % 2]).wait()

        def sum_step(i, a):
            o = i * (2 * L)
            return (a[0] + bset[pl.ds(o, L)], a[1] + bset[pl.ds(o + L, L)],
                    a[2], a[3])
        accs = list(lax.fori_loop(0, SET // (2 * L), sum_step, tuple(accs)))
    acc = (accs[0] + accs[1]) + (accs[2] + accs[3])

    pvt_v[...] = acc
    pltpu.sync_copy(pvt_v, part_sp.at[sid])
    h_stage.wait()

    plsc.subcore_barrier()

    h_idx.wait()
    sems_g = (sem_g0, sem_g1)

    def fire_group(g):
        s = sems_g[g % 2]

        def fire(r, c):
            row = g * G + r
            pltpu.async_copy(table_sp.at[idx_v.at[row]], vals_v.at[row], s)
            return c
        lax.fori_loop(0, G, fire, 0)

    fire_group(0)
    fire_group(1)

    pltpu.sync_copy(part_sp, pall_v)
    tot = pall_v[0]
    for t in range(1, NS):
        tot = tot + pall_v[t]
    s_scalar = tot[0]
    for i in range(1, L):
        s_scalar = s_scalar + tot[i]
    ln_s = _log_from_bits(jnp.full((L,), s_scalar, jnp.float32),
                          jnp.zeros((L,), jnp.float32))

    for g in range(NG):
        pltpu.make_async_copy(
            out_hbm.at[pl.ds(obase + g * G, G)],
            vals_v.at[pl.ds(g * G, G)], sems_g[g % 2]).wait()
        if g + 2 < NG:
            fire_group(g + 2)

        def log_row(r, carry):
            row = g * G + r
            for c in range(128 // L):
                v = vals_v[row, pl.ds(c * L, L)]
                vals_v[row, pl.ds(c * L, L)] = _log_from_bits(v, ln_s)
            return carry
        lax.fori_loop(0, G, log_row, 0)

        pltpu.async_copy(vals_v.at[pl.ds(g * G, G)],
                         out_hbm.at[pl.ds(obase + g * G, G)], sem_out)

    pltpu.make_async_copy(out_hbm.at[pl.ds(obase, ROWS_W)],
                          vals_v, sem_out).wait()


@jax.jit
def kernel(indices, probs):
    idx2d = indices.reshape(ROWS, 128)
    probs_pad = jnp.concatenate(
        [probs, jnp.zeros((VPAD - NUM_CLASSES,), jnp.float32)])
    mesh = plsc.VectorSubcoreMesh(core_axis_name="c", subcore_axis_name="s")
    fn = pl.kernel(
        _body,
        out_type=jax.ShapeDtypeStruct((ROWS, 128), jnp.float32),
        mesh=mesh,
        scratch_types=[
            pltpu.VMEM_SHARED((VPAD,), jnp.float32),
            pltpu.VMEM_SHARED((NS, L), jnp.float32),
            pltpu.VMEM((SET,), jnp.float32),
            pltpu.VMEM((SET,), jnp.float32),
            pltpu.VMEM((ROWS_W, 128), jnp.int32),
            pltpu.VMEM((ROWS_W, 128), jnp.float32),
            pltpu.VMEM((L,), jnp.float32),
            pltpu.VMEM((NS, L), jnp.float32),
        ] + [pltpu.SemaphoreType.DMA] * 7,
    )
    out = fn(idx2d, probs_pad)
    return out.reshape(4096, 200)


# speedup vs baseline: 1.8103x; 1.0569x over previous
import jax
import jax.numpy as jnp
from jax import lax
from jax.experimental import pallas as pl
from jax.experimental.pallas import tpu as pltpu
from jax.experimental.pallas import tpu_sc as plsc

V = 1_000_000
NC = 2
NS = 16
L = 16
NW = NC * NS

CHUNK = 62_976
VPAD = NS * CHUNK
NWAVE = 12
WAVE = CHUNK // NWAVE
SUM_U = 8

B_TOT = 4096 * 200
ROWS = B_TOT // 128
ROWS_W = ROWS // NW
G = 40
NG = ROWS_W // G

_LN2 = 0.6931471805599453
_P5 = (0.0439291, -0.40947993, 1.61019286, -3.52024492, 5.06977785, -2.7941606)


def _log_from_bits(v, ln_s):
    bits = lax.bitcast_convert_type(v, jnp.int32)
    e = lax.shift_right_arithmetic(bits, 23) - 127
    m_bits = lax.bitwise_or(lax.bitwise_and(bits, 0x007FFFFF), 0x3F800000)
    m = lax.bitcast_convert_type(m_bits, jnp.float32)
    p = jnp.float32(_P5[0])
    for c in _P5[1:]:
        p = p * m + jnp.float32(c)
    return (e.astype(jnp.float32) + p) * jnp.float32(_LN2) - ln_s


def _body(idx_hbm, probs_hbm, out_hbm,
          table_sp, part_sp, buf_a, buf_b, idx_v, vals_v, pvt_v, pall_v,
          sem_a, sem_b, sem_stage, sem_idx, sem_g0, sem_g1, sem_out):
    cid = lax.axis_index("c")
    sid = lax.axis_index("s")
    wid = sid * NC + cid
    obase = wid * ROWS_W

    h_idx = pltpu.async_copy(idx_hbm.at[pl.ds(obase, ROWS_W)], idx_v, sem_idx)

    base = sid * CHUNK
    pltpu.async_copy(probs_hbm.at[pl.ds(base, CHUNK)],
                     table_sp.at[pl.ds(base, CHUNK)], sem_stage).wait()

    plsc.subcore_barrier()

    h_idx.wait()
    sems_g = (sem_g0, sem_g1)

    def fire_group(g):
        s = sems_g[g % 2]

        def fire(r, c):
            row = g * G + r
            pltpu.async_copy(table_sp.at[idx_v.at[row]], vals_v.at[row], s)
            return c
        lax.fori_loop(0, G, fire, 0)

    fire_group(0)
    fire_group(1)

    sbase = base
    sets = (buf_a, buf_b)
    sems = (sem_a, sem_b)

    def fire_wave(w):
        pltpu.async_copy(
            table_sp.at[pl.ds(sbase + w * WAVE, WAVE)],
            sets[w % 2], sems[w % 2])

    fire_wave(0)
    accs = [jnp.zeros((L,), jnp.float32) for _ in range(4)]
    for w in range(NWAVE):
        if w + 1 < NWAVE:
            fire_wave(w + 1)
        bset = sets[w % 2]
        pltpu.make_async_copy(
            probs_hbm.at[pl.ds(0, WAVE)], bset, sems[w % 2]).wait()

        def sum_step(i, a):
            o = i * (SUM_U * L)
            for u in range(SUM_U):
                a = tuple(
                    a[j] + bset[pl.ds(o + u * L, L)] if j == (u % 4) else a[j]
                    for j in range(4))
            return a
        accs = list(lax.fori_loop(0, WAVE // (SUM_U * L), sum_step,
                                  tuple(accs)))

    acc = (accs[0] + accs[1]) + (accs[2] + accs[3])
    pvt_v[...] = acc
    pltpu.sync_copy(pvt_v, part_sp.at[sid])

    plsc.subcore_barrier()

    pltpu.sync_copy(part_sp, pall_v)
    tot = pall_v[0]
    for t in range(1, NS):
        tot = tot + pall_v[t]
    s_scalar = tot[0]
    for i in range(1, L):
        s_scalar = s_scalar + tot[i]
    ln_s = _log_from_bits(jnp.full((L,), s_scalar, jnp.float32),
                          jnp.zeros((L,), jnp.float32))

    for g in range(NG):
        pltpu.make_async_copy(
            out_hbm.at[pl.ds(obase + g * G, G)],
            vals_v.at[pl.ds(g * G, G)], sems_g[g % 2]).wait()
        if g + 2 < NG:
            fire_group(g + 2)

        def log_row(r, carry):
            row = g * G + r
            for c in range(128 // L):
                v = vals_v[row, pl.ds(c * L, L)]
                vals_v[row, pl.ds(c * L, L)] = _log_from_bits(v, ln_s)
            return carry
        lax.fori_loop(0, G, log_row, 0)

        pltpu.async_copy(vals_v.at[pl.ds(g * G, G)],
                         out_hbm.at[pl.ds(obase + g * G, G)], sem_out)

    pltpu.make_async_copy(out_hbm.at[pl.ds(obase, ROWS_W)],
                          vals_v, sem_out).wait()


@jax.jit
def kernel(indices, probs):
    idx2d = indices.reshape(ROWS, 128)
    probs_pad = jnp.pad(probs, (0, VPAD - V))
    mesh = plsc.VectorSubcoreMesh(core_axis_name="c", subcore_axis_name="s")
    fn = pl.kernel(
        _body,
        out_type=jax.ShapeDtypeStruct((ROWS, 128), jnp.float32),
        mesh=mesh,
        scratch_types=[
            pltpu.VMEM_SHARED((VPAD,), jnp.float32),
            pltpu.VMEM_SHARED((NS, L), jnp.float32),
            pltpu.VMEM((WAVE,), jnp.float32),
            pltpu.VMEM((WAVE,), jnp.float32),
            pltpu.VMEM((ROWS_W, 128), jnp.int32),
            pltpu.VMEM((ROWS_W, 128), jnp.float32),
            pltpu.VMEM((L,), jnp.float32),
            pltpu.VMEM((NS, L), jnp.float32),
        ] + [pltpu.SemaphoreType.DMA] * 7,
    )
    out = fn(idx2d, probs_pad)
    return out.reshape(4096, 200)


# speedup vs baseline: 1.8464x; 1.0199x over previous
import jax
import jax.numpy as jnp
from jax import lax
from jax.experimental import pallas as pl
from jax.experimental.pallas import tpu as pltpu
from jax.experimental.pallas import tpu_sc as plsc

V = 1_000_000
NC = 2
NS = 16
L = 16
NW = NC * NS

CHUNK = 62_976
VPAD = NS * CHUNK
NWAVE = 12
WAVE = CHUNK // NWAVE
SUM_U = 8

B_TOT = 4096 * 200
ROWS = B_TOT // 128
ROWS_W = ROWS // NW
G = 40
NG = ROWS_W // G

_LN2 = 0.6931471805599453
_P4 = (-0.05545986968073571, 0.44050704554227527, -1.4552065437591728,
       2.806994158628966, -1.7367654165499555)


def _log_from_bits(v, ln_s):
    bits = lax.bitcast_convert_type(v, jnp.int32)
    e = lax.shift_right_arithmetic(bits, 23) - 127
    m_bits = lax.bitwise_or(lax.bitwise_and(bits, 0x007FFFFF), 0x3F800000)
    m = lax.bitcast_convert_type(m_bits, jnp.float32)
    p = jnp.float32(_P4[0])
    for c in _P4[1:-1]:
        p = p * m + jnp.float32(c)
    p = p * m + (jnp.float32(_P4[-1]) - ln_s)
    return e.astype(jnp.float32) * jnp.float32(_LN2) + p


def _body(idx_hbm, probs_hbm, out_hbm,
          table_sp, part_sp, buf_a, buf_b, idx_v, vals_v, pvt_v, pall_v,
          sem_a, sem_b, sem_stage, sem_idx, sem_g0, sem_g1, sem_out):
    cid = lax.axis_index("c")
    sid = lax.axis_index("s")
    wid = sid * NC + cid
    obase = wid * ROWS_W

    h_idx = pltpu.async_copy(idx_hbm.at[pl.ds(obase, ROWS_W)], idx_v, sem_idx)

    base = sid * CHUNK
    pltpu.async_copy(probs_hbm.at[pl.ds(base, CHUNK)],
                     table_sp.at[pl.ds(base, CHUNK)], sem_stage).wait()

    plsc.subcore_barrier()

    h_idx.wait()
    sems_g = (sem_g0, sem_g1)

    def fire_group(g):
        s = sems_g[g % 2]

        def fire(r, c):
            row = g * G + r
            pltpu.async_copy(table_sp.at[idx_v.at[row]], vals_v.at[row], s)
            return c
        lax.fori_loop(0, G, fire, 0)

    fire_group(0)
    fire_group(1)

    sbase = base
    sets = (buf_a, buf_b)
    sems = (sem_a, sem_b)

    def fire_wave(w):
        pltpu.async_copy(
            table_sp.at[pl.ds(sbase + w * WAVE, WAVE)],
            sets[w % 2], sems[w % 2])

    fire_wave(0)
    accs = [jnp.zeros((L,), jnp.float32) for _ in range(4)]
    for w in range(NWAVE):
        if w + 1 < NWAVE:
            fire_wave(w + 1)
        bset = sets[w % 2]
        pltpu.make_async_copy(
            probs_hbm.at[pl.ds(0, WAVE)], bset, sems[w % 2]).wait()

        def sum_step(i, a):
            o = i * (SUM_U * L)
            for u in range(SUM_U):
                a = tuple(
                    a[j] + bset[pl.ds(o + u * L, L)] if j == (u % 4) else a[j]
                    for j in range(4))
            return a
        accs = list(lax.fori_loop(0, WAVE // (SUM_U * L), sum_step,
                                  tuple(accs)))

    acc = (accs[0] + accs[1]) + (accs[2] + accs[3])
    pvt_v[...] = acc
    pltpu.sync_copy(pvt_v, part_sp.at[sid])

    plsc.subcore_barrier()

    pltpu.sync_copy(part_sp, pall_v)
    tot = pall_v[0]
    for t in range(1, NS):
        tot = tot + pall_v[t]
    s_scalar = tot[0]
    for i in range(1, L):
        s_scalar = s_scalar + tot[i]
    ln_s = _log_from_bits(jnp.full((L,), s_scalar, jnp.float32),
                          jnp.zeros((L,), jnp.float32))

    for g in range(NG):
        pltpu.make_async_copy(
            out_hbm.at[pl.ds(obase + g * G, G)],
            vals_v.at[pl.ds(g * G, G)], sems_g[g % 2]).wait()
        if g + 2 < NG:
            fire_group(g + 2)

        def log_row(r, carry):
            row = g * G + r
            for c in range(128 // L):
                v = vals_v[row, pl.ds(c * L, L)]
                vals_v[row, pl.ds(c * L, L)] = _log_from_bits(v, ln_s)
            return carry
        lax.fori_loop(0, G, log_row, 0)

        pltpu.async_copy(vals_v.at[pl.ds(g * G, G)],
                         out_hbm.at[pl.ds(obase + g * G, G)], sem_out)

    pltpu.make_async_copy(out_hbm.at[pl.ds(obase, ROWS_W)],
                          vals_v, sem_out).wait()


@jax.jit
def kernel(indices, probs):
    idx2d = indices.reshape(ROWS, 128)
    probs_pad = jnp.pad(probs, (0, VPAD - V))
    mesh = plsc.VectorSubcoreMesh(core_axis_name="c", subcore_axis_name="s")
    fn = pl.kernel(
        _body,
        out_type=jax.ShapeDtypeStruct((ROWS, 128), jnp.float32),
        mesh=mesh,
        scratch_types=[
            pltpu.VMEM_SHARED((VPAD,), jnp.float32),
            pltpu.VMEM_SHARED((NS, L), jnp.float32),
            pltpu.VMEM((WAVE,), jnp.float32),
            pltpu.VMEM((WAVE,), jnp.float32),
            pltpu.VMEM((ROWS_W, 128), jnp.int32),
            pltpu.VMEM((ROWS_W, 128), jnp.float32),
            pltpu.VMEM((L,), jnp.float32),
            pltpu.VMEM((NS, L), jnp.float32),
        ] + [pltpu.SemaphoreType.DMA] * 7,
    )
    out = fn(idx2d, probs_pad)
    return out.reshape(4096, 200)
